# baseline (speedup 1.0000x reference)
import jax
import jax.numpy as jnp
from jax import lax
from jax.experimental import pallas as pl
from jax.experimental.pallas import tpu as pltpu
from jax.experimental.pallas import tpu_sc as plsc

N = 10000
E = 160000
F = 256
QF = 64
NSC = 2
NTEC = 16
LANES = 16

N_PAD = 10240
ROWS_PER_TEC = N_PAD // NTEC
EDGE_BLK = 128
EPT = 10240
NB = EPT // EDGE_BLK
E_PAD = EPT * NTEC
NRING = 4

BN = 1000


def _sc_degree_body(src_hbm, dst_hbm, out_hbm, idx_v, hist_v):
    c = lax.axis_index("c")
    s = lax.axis_index("s")

    @pl.loop(0, N_PAD // LANES)
    def _zero(i):
        hist_v[pl.ds(i * LANES, LANES)] = jnp.zeros((LANES,), jnp.float32)

    @pl.when(c == 0)
    def _():
        pltpu.sync_copy(src_hbm.at[s], idx_v)

    @pl.when(c == 1)
    def _():
        pltpu.sync_copy(dst_hbm.at[s], idx_v)

    ones = jnp.ones((LANES,), jnp.float32)

    @pl.loop(0, NB)
    def _blocks(b):
        for j in range(EDGE_BLK // LANES):
            idx = idx_v[b, pl.ds(j * LANES, LANES)]
            plsc.addupdate_scatter(hist_v, [idx], ones)

    pltpu.sync_copy(hist_v, out_hbm.at[c, s])


def _sc_degree(src_p, dst_p):
    mesh = plsc.VectorSubcoreMesh(core_axis_name="c", subcore_axis_name="s")
    return pl.kernel(
        _sc_degree_body,
        out_type=jax.ShapeDtypeStruct((NSC, NTEC, N_PAD), jnp.float32),
        mesh=mesh,
        scratch_types=[
            pltpu.VMEM((NB, EDGE_BLK), jnp.int32),
            pltpu.VMEM((N_PAD,), jnp.float32),
        ],
        compiler_params=pltpu.CompilerParams(needs_layout_passes=False),
    )(src_p, dst_p)


def _sc_agg_body(hs_q0, hs_q1, hs_q2, hs_q3, src_hbm, dst_hbm,
                 out_q0, out_q1, out_q2, out_q3,
                 idx_s, idx_d, rows_v, zero_v, agg_sh,
                 gsem0, gsem1, gsem2, gsem3, ssem0, ssem1, ssem2, ssem3):
    c = lax.axis_index("c")
    s = lax.axis_index("s")
    gsems = (gsem0, gsem1, gsem2, gsem3)
    ssems = (ssem0, ssem1, ssem2, ssem3)

    pltpu.sync_copy(src_hbm.at[s], idx_s)
    pltpu.sync_copy(dst_hbm.at[s], idx_d)

    for r in range(64):
        for j in range(QF // LANES):
            zero_v[r, pl.ds(j * LANES, LANES)] = jnp.zeros((LANES,), jnp.float32)

    def one_pass(table, out):
        for r in range(ROWS_PER_TEC // 64):
            pltpu.sync_copy(zero_v,
                            agg_sh.at[pl.ds(s * ROWS_PER_TEC + r * 64, 64)])
        plsc.subcore_barrier()

        for k in range(NRING):
            pltpu.async_copy(table.at[idx_s.at[k]], rows_v.at[k], gsems[k])

        @pl.loop(0, NB, step=NRING)
        def _blk(b):
            for p in range(NRING):
                j = b + p
                pltpu.make_async_copy(table.at[idx_s.at[j]], rows_v.at[p],
                                      gsems[p]).wait()
                pltpu.async_copy(rows_v.at[p], agg_sh.at[idx_d.at[j]],
                                 ssems[p], add=True)
                pn = (p + 1) % NRING

                @pl.when(jnp.logical_and(j >= NRING - 1, j + 1 < NB))
                def _():
                    pltpu.make_async_copy(rows_v.at[pn],
                                          agg_sh.at[idx_d.at[j - NRING + 1]],
                                          ssems[pn]).wait()
                    pltpu.async_copy(table.at[idx_s.at[j + 1]],
                                     rows_v.at[pn], gsems[pn])

        for p in range(NRING):
            pltpu.make_async_copy(rows_v.at[p],
                                  agg_sh.at[idx_d.at[NB - NRING + p]],
                                  ssems[p]).wait()
        plsc.subcore_barrier()
        off = s * ROWS_PER_TEC
        pltpu.sync_copy(agg_sh.at[pl.ds(off, ROWS_PER_TEC)],
                        out.at[pl.ds(off, ROWS_PER_TEC)])
        plsc.subcore_barrier()

    @pl.when(c == 0)
    def _():
        one_pass(hs_q0, out_q0)
        one_pass(hs_q1, out_q1)

    @pl.when(c == 1)
    def _():
        one_pass(hs_q2, out_q2)
        one_pass(hs_q3, out_q3)


def _sc_agg(hs_q, src_p, dst_p):
    mesh = plsc.VectorSubcoreMesh(core_axis_name="c", subcore_axis_name="s")
    qshape = jax.ShapeDtypeStruct((N_PAD, QF), jnp.float32)
    return pl.kernel(
        _sc_agg_body,
        out_type=(qshape,) * 4,
        mesh=mesh,
        scratch_types=[
            pltpu.VMEM((NB, EDGE_BLK), jnp.int32),
            pltpu.VMEM((NB, EDGE_BLK), jnp.int32),
            pltpu.VMEM((NRING, EDGE_BLK, QF), jnp.float32),
            pltpu.VMEM((64, QF), jnp.float32),
            pltpu.VMEM_SHARED((N_PAD, QF), jnp.float32),
            pltpu.SemaphoreType.DMA,
            pltpu.SemaphoreType.DMA,
            pltpu.SemaphoreType.DMA,
            pltpu.SemaphoreType.DMA,
            pltpu.SemaphoreType.DMA,
            pltpu.SemaphoreType.DMA,
            pltpu.SemaphoreType.DMA,
            pltpu.SemaphoreType.DMA,
        ],
        compiler_params=pltpu.CompilerParams(needs_layout_passes=False,
                                             use_tc_tiling_on_sc=False),
    )(*hs_q, src_p, dst_p)


def _split_store(hs, outs):
    for q in range(4):
        outs[q][...] = hs[:, q * QF:(q + 1) * QF]


def _tc_embed_body(x_ref, w_ref, hist_ref, *outs):
    h = jnp.dot(x_ref[...], w_ref[...], preferred_element_type=jnp.float32)
    deg = jnp.sum(hist_ref[0], axis=1, keepdims=True)
    ns = lax.rsqrt(jnp.maximum(deg, 1.0))
    _split_store(h * ns, outs)


_HSPECS = [pl.BlockSpec((BN, QF), lambda i: (i, 0)) for _ in range(4)]
_HSHAPES = [jax.ShapeDtypeStruct((N_PAD, QF), jnp.float32) for _ in range(4)]


def _tc_embed(x, w0, hists_t):
    return pl.pallas_call(
        _tc_embed_body,
        grid=(N // BN,),
        in_specs=[
            pl.BlockSpec((BN, F), lambda i: (i, 0)),
            pl.BlockSpec((F, F), lambda i: (0, 0)),
            pl.BlockSpec((NSC, BN, NTEC), lambda i: (0, i, 0)),
        ],
        out_specs=_HSPECS,
        out_shape=_HSHAPES,
    )(x, w0, hists_t)


def _layer_common(aggs, hist_ref, w_ref, b_ref):
    a = jnp.concatenate([q[...] for q in aggs], axis=1)
    deg_in = jnp.sum(hist_ref[1], axis=1, keepdims=True)
    nd = lax.rsqrt(jnp.maximum(deg_in, 1.0))
    h = jnp.dot(a * nd, w_ref[...], preferred_element_type=jnp.float32)
    return jnp.maximum(h + b_ref[...], 0.0)


def _tc_layer_body(a0, a1, a2, a3, hist_ref, w_ref, b_ref, *outs):
    h = _layer_common((a0, a1, a2, a3), hist_ref, w_ref, b_ref)
    deg_out = jnp.sum(hist_ref[0], axis=1, keepdims=True)
    ns = lax.rsqrt(jnp.maximum(deg_out, 1.0))
    _split_store(h * ns, outs)


def _tc_layer(aggs, hists_t, w, b):
    return pl.pallas_call(
        _tc_layer_body,
        grid=(N // BN,),
        in_specs=_HSPECS + [
            pl.BlockSpec((NSC, BN, NTEC), lambda i: (0, i, 0)),
            pl.BlockSpec((F, F), lambda i: (0, 0)),
            pl.BlockSpec((1, F), lambda i: (0, 0)),
        ],
        out_specs=_HSPECS,
        out_shape=_HSHAPES,
    )(*aggs, hists_t, w, b)


def _tc_final_body(a0, a1, a2, a3, hist_ref, w_ref, b_ref, out_ref):
    i = pl.program_id(0)

    @pl.when(i == 0)
    def _():
        out_ref[...] = jnp.full((1, F), -jnp.inf, jnp.float32)

    h = _layer_common((a0, a1, a2, a3), hist_ref, w_ref, b_ref)
    out_ref[...] = jnp.maximum(out_ref[...], jnp.max(h, axis=0, keepdims=True))


def _tc_final(aggs, hists_t, w, b):
    return pl.pallas_call(
        _tc_final_body,
        grid=(N // BN,),
        in_specs=_HSPECS + [
            pl.BlockSpec((NSC, BN, NTEC), lambda i: (0, i, 0)),
            pl.BlockSpec((F, F), lambda i: (0, 0)),
            pl.BlockSpec((1, F), lambda i: (0, 0)),
        ],
        out_specs=pl.BlockSpec((1, F), lambda i: (0, 0)),
        out_shape=jax.ShapeDtypeStruct((1, F), jnp.float32),
        compiler_params=pltpu.CompilerParams(
            dimension_semantics=("arbitrary",)),
    )(*aggs, hists_t, w, b)


@jax.jit
def kernel(x, edge_index, W0, W1, b1, W2, b2):
    src = edge_index[0]
    dst = edge_index[1]
    pad = jnp.full((E_PAD - E,), N, jnp.int32)
    src_p = jnp.concatenate([src, pad]).reshape(NTEC, NB, EDGE_BLK)
    dst_p = jnp.concatenate([dst, pad]).reshape(NTEC, NB, EDGE_BLK)

    hists = _sc_degree(src_p, dst_p)
    hists_t = hists.transpose(0, 2, 1)

    hs_h = _tc_embed(x, W0, hists_t)
    agg_h = _sc_agg(hs_h, src_p, dst_p)
    hs_h = _tc_layer(agg_h, hists_t, W1, b1.reshape(1, F))
    agg_h = _sc_agg(hs_h, src_p, dst_p)
    out = _tc_final(agg_h, hists_t, W2, b2.reshape(1, F))
    return out.reshape(F)


# speedup vs baseline: 1.1940x; 1.1940x over previous
import jax
import jax.numpy as jnp
from jax import lax
from jax.experimental import pallas as pl
from jax.experimental.pallas import tpu as pltpu
from jax.experimental.pallas import tpu_sc as plsc

N = 10000
E = 160000
F = 256
QF = 64
NSC = 2
NTEC = 16
LANES = 16

N_PAD = 10240
ROWS_PER_TEC = N_PAD // NTEC
EDGE_BLK = 128
EPT = 10112
NB = EPT // EDGE_BLK
E_PAD = EPT * NTEC

BN = 1000


def _sc_degree_body(src_hbm, dst_hbm, out_hbm, idx_v, hist_v, sum_v, deg_v,
                    hist_sh):
    c = lax.axis_index("c")
    s = lax.axis_index("s")

    @pl.loop(0, N_PAD // LANES)
    def _zero(i):
        hist_v[pl.ds(i * LANES, LANES)] = jnp.zeros((LANES,), jnp.float32)

    @pl.when(c == 0)
    def _():
        pltpu.sync_copy(src_hbm.at[s], idx_v)

    @pl.when(c == 1)
    def _():
        pltpu.sync_copy(dst_hbm.at[s], idx_v)

    ones = jnp.ones((LANES,), jnp.float32)

    @pl.loop(0, NB)
    def _blocks(b):
        for j in range(EDGE_BLK // LANES):
            idx = idx_v[b, pl.ds(j * LANES, LANES)]
            plsc.addupdate_scatter(hist_v, [idx], ones)

    pltpu.sync_copy(hist_v, hist_sh.at[s])
    plsc.subcore_barrier()
    pltpu.sync_copy(hist_sh.at[:, pl.ds(s * ROWS_PER_TEC, ROWS_PER_TEC)],
                    sum_v)
    for k in range(ROWS_PER_TEC // LANES):
        acc = jnp.zeros((LANES,), jnp.float32)
        for t in range(NTEC):
            acc = acc + sum_v[t, pl.ds(k * LANES, LANES)]
        deg_v[pl.ds(k * LANES, LANES)] = acc
    pltpu.sync_copy(deg_v, out_hbm.at[c, pl.ds(s * ROWS_PER_TEC,
                                               ROWS_PER_TEC)])


def _sc_degree(src_p, dst_p):
    mesh = plsc.VectorSubcoreMesh(core_axis_name="c", subcore_axis_name="s")
    return pl.kernel(
        _sc_degree_body,
        out_type=jax.ShapeDtypeStruct((NSC, N_PAD), jnp.float32),
        mesh=mesh,
        scratch_types=[
            pltpu.VMEM((NB, EDGE_BLK), jnp.int32),
            pltpu.VMEM((N_PAD,), jnp.float32),
            pltpu.VMEM((NTEC, ROWS_PER_TEC), jnp.float32),
            pltpu.VMEM((ROWS_PER_TEC,), jnp.float32),
            pltpu.VMEM_SHARED((NTEC, N_PAD), jnp.float32),
        ],
        compiler_params=pltpu.CompilerParams(needs_layout_passes=False),
    )(src_p, dst_p)


def _sc_agg_body(hs_q0, hs_q1, hs_q2, hs_q3, src_hbm, dst_hbm,
                 out_q0, out_q1, out_q2, out_q3,
                 idx_s, idx_d, rows_v, zero_v, agg_sh, gsem):
    c = lax.axis_index("c")
    s = lax.axis_index("s")

    pltpu.sync_copy(src_hbm.at[s], idx_s)
    pltpu.sync_copy(dst_hbm.at[s], idx_d)

    for r in range(64):
        for j in range(QF // LANES):
            zero_v[r, pl.ds(j * LANES, LANES)] = jnp.zeros((LANES,), jnp.float32)

    def one_pass(table, out):
        for r in range(ROWS_PER_TEC // 64):
            pltpu.sync_copy(zero_v,
                            agg_sh.at[pl.ds(s * ROWS_PER_TEC + r * 64, 64)])
        plsc.subcore_barrier()

        @pl.loop(0, NB)
        def _blocks(b):
            pltpu.async_copy(table.at[idx_s.at[b]], rows_v, gsem).wait()
            pltpu.sync_copy(rows_v, agg_sh.at[idx_d.at[b]], add=True)

        plsc.subcore_barrier()
        off = s * ROWS_PER_TEC
        pltpu.sync_copy(agg_sh.at[pl.ds(off, ROWS_PER_TEC)],
                        out.at[pl.ds(off, ROWS_PER_TEC)])
        plsc.subcore_barrier()

    @pl.when(c == 0)
    def _():
        one_pass(hs_q0, out_q0)
        one_pass(hs_q1, out_q1)

    @pl.when(c == 1)
    def _():
        one_pass(hs_q2, out_q2)
        one_pass(hs_q3, out_q3)


def _sc_agg(hs_q, src_p, dst_p):
    mesh = plsc.VectorSubcoreMesh(core_axis_name="c", subcore_axis_name="s")
    qshape = jax.ShapeDtypeStruct((N_PAD, QF), jnp.float32)
    return pl.kernel(
        _sc_agg_body,
        out_type=(qshape,) * 4,
        mesh=mesh,
        scratch_types=[
            pltpu.VMEM((NB, EDGE_BLK), jnp.int32),
            pltpu.VMEM((NB, EDGE_BLK), jnp.int32),
            pltpu.VMEM((EDGE_BLK, QF), jnp.float32),
            pltpu.VMEM((64, QF), jnp.float32),
            pltpu.VMEM_SHARED((N_PAD, QF), jnp.float32),
            pltpu.SemaphoreType.DMA,
        ],
        compiler_params=pltpu.CompilerParams(needs_layout_passes=False,
                                             use_tc_tiling_on_sc=False),
    )(*hs_q, src_p, dst_p)


def _split_store(hs, outs):
    for q in range(4):
        outs[q][...] = hs[:, q * QF:(q + 1) * QF]


def _tc_embed_body(x_ref, w_ref, deg_ref, *outs):
    h = jnp.dot(x_ref[...], w_ref[...], preferred_element_type=jnp.float32)
    ns = lax.rsqrt(jnp.maximum(deg_ref[0], 1.0))
    _split_store(h * ns, outs)


_HSPECS = [pl.BlockSpec((BN, QF), lambda i: (i, 0)) for _ in range(4)]
_HSHAPES = [jax.ShapeDtypeStruct((N_PAD, QF), jnp.float32) for _ in range(4)]


def _tc_embed(x, w0, degc):
    return pl.pallas_call(
        _tc_embed_body,
        grid=(N // BN,),
        in_specs=[
            pl.BlockSpec((BN, F), lambda i: (i, 0)),
            pl.BlockSpec((F, F), lambda i: (0, 0)),
            pl.BlockSpec((NSC, BN, 1), lambda i: (0, i, 0)),
        ],
        out_specs=_HSPECS,
        out_shape=_HSHAPES,
    )(x, w0, degc)


def _layer_common(aggs, deg_ref, w_ref, b_ref):
    a = jnp.concatenate([q[...] for q in aggs], axis=1)
    nd = lax.rsqrt(jnp.maximum(deg_ref[1], 1.0))
    h = jnp.dot(a * nd, w_ref[...], preferred_element_type=jnp.float32)
    return jnp.maximum(h + b_ref[...], 0.0)


def _tc_layer_body(a0, a1, a2, a3, deg_ref, w_ref, b_ref, *outs):
    h = _layer_common((a0, a1, a2, a3), deg_ref, w_ref, b_ref)
    ns = lax.rsqrt(jnp.maximum(deg_ref[0], 1.0))
    _split_store(h * ns, outs)


def _tc_layer(aggs, degc, w, b):
    return pl.pallas_call(
        _tc_layer_body,
        grid=(N // BN,),
        in_specs=_HSPECS + [
            pl.BlockSpec((NSC, BN, 1), lambda i: (0, i, 0)),
            pl.BlockSpec((F, F), lambda i: (0, 0)),
            pl.BlockSpec((1, F), lambda i: (0, 0)),
        ],
        out_specs=_HSPECS,
        out_shape=_HSHAPES,
    )(*aggs, degc, w, b)


def _tc_final_body(a0, a1, a2, a3, deg_ref, w_ref, b_ref, out_ref):
    i = pl.program_id(0)

    @pl.when(i == 0)
    def _():
        out_ref[...] = jnp.full((1, F), -jnp.inf, jnp.float32)

    h = _layer_common((a0, a1, a2, a3), deg_ref, w_ref, b_ref)
    out_ref[...] = jnp.maximum(out_ref[...], jnp.max(h, axis=0, keepdims=True))


def _tc_final(aggs, degc, w, b):
    return pl.pallas_call(
        _tc_final_body,
        grid=(N // BN,),
        in_specs=_HSPECS + [
            pl.BlockSpec((NSC, BN, 1), lambda i: (0, i, 0)),
            pl.BlockSpec((F, F), lambda i: (0, 0)),
            pl.BlockSpec((1, F), lambda i: (0, 0)),
        ],
        out_specs=pl.BlockSpec((1, F), lambda i: (0, 0)),
        out_shape=jax.ShapeDtypeStruct((1, F), jnp.float32),
        compiler_params=pltpu.CompilerParams(
            dimension_semantics=("arbitrary",)),
    )(*aggs, degc, w, b)


@jax.jit
def kernel(x, edge_index, W0, W1, b1, W2, b2):
    src = edge_index[0]
    dst = edge_index[1]
    pad = jnp.full((E_PAD - E,), N, jnp.int32)
    src_p = jnp.concatenate([src, pad]).reshape(NTEC, NB, EDGE_BLK)
    dst_p = jnp.concatenate([dst, pad]).reshape(NTEC, NB, EDGE_BLK)

    deg = _sc_degree(src_p, dst_p)
    degc = deg.reshape(NSC, N_PAD, 1)

    hs_q = _tc_embed(x, W0, degc)
    agg_q = _sc_agg(hs_q, src_p, dst_p)
    hs_q = _tc_layer(agg_q, degc, W1, b1.reshape(1, F))
    agg_q = _sc_agg(hs_q, src_p, dst_p)
    out = _tc_final(agg_q, degc, W2, b2.reshape(1, F))
    return out.reshape(F)


# speedup vs baseline: 1.2368x; 1.0359x over previous
import jax
import jax.numpy as jnp
from jax import lax
from jax.experimental import pallas as pl
from jax.experimental.pallas import tpu as pltpu
from jax.experimental.pallas import tpu_sc as plsc

N = 10000
E = 160000
F = 256
QF = 64
NSC = 2
NTEC = 16
LANES = 16

N_PAD = 10240
ROWS_PER_TEC = N_PAD // NTEC
EDGE_BLK = 128
EPT = 10112
NB = EPT // EDGE_BLK
E_PAD = EPT * NTEC

BN = 1000


def _sc_degree_body(src_hbm, dst_hbm, out_hbm, idx_v, hist_v):
    c = lax.axis_index("c")
    s = lax.axis_index("s")

    @pl.loop(0, N_PAD // LANES)
    def _zero(i):
        hist_v[pl.ds(i * LANES, LANES)] = jnp.zeros((LANES,), jnp.float32)

    @pl.when(c == 0)
    def _():
        pltpu.sync_copy(src_hbm.at[s], idx_v)

    @pl.when(c == 1)
    def _():
        pltpu.sync_copy(dst_hbm.at[s], idx_v)

    ones = jnp.ones((LANES,), jnp.float32)

    @pl.loop(0, NB)
    def _blocks(b):
        for j in range(EDGE_BLK // LANES):
            idx = idx_v[b, pl.ds(j * LANES, LANES)]
            plsc.addupdate_scatter(hist_v, [idx], ones)

    pltpu.sync_copy(hist_v, out_hbm.at[c, s])


def _sc_degree(src_p, dst_p):
    mesh = plsc.VectorSubcoreMesh(core_axis_name="c", subcore_axis_name="s")
    return pl.kernel(
        _sc_degree_body,
        out_type=jax.ShapeDtypeStruct((NSC, NTEC, N_PAD), jnp.float32),
        mesh=mesh,
        scratch_types=[
            pltpu.VMEM((NB, EDGE_BLK), jnp.int32),
            pltpu.VMEM((N_PAD,), jnp.float32),
        ],
        compiler_params=pltpu.CompilerParams(needs_layout_passes=False),
    )(src_p, dst_p)


def _sc_agg_body(hs_q0, hs_q1, hs_q2, hs_q3, src_hbm, dst_hbm,
                 out_q0, out_q1, out_q2, out_q3,
                 idx_s, idx_d, rows_v, zero_v, agg_sh, gsem):
    c = lax.axis_index("c")
    s = lax.axis_index("s")

    for r in range(64):
        for j in range(QF // LANES):
            zero_v[r, pl.ds(j * LANES, LANES)] = jnp.zeros((LANES,), jnp.float32)

    pltpu.sync_copy(src_hbm.at[s], idx_s)
    pltpu.sync_copy(dst_hbm.at[s], idx_d)

    def one_pass(table, out):
        for r in range(ROWS_PER_TEC // 64):
            pltpu.sync_copy(zero_v,
                            agg_sh.at[pl.ds(s * ROWS_PER_TEC + r * 64, 64)])
        plsc.subcore_barrier()

        @pl.loop(0, NB)
        def _blocks(b):
            pltpu.async_copy(table.at[idx_s.at[b]], rows_v, gsem).wait()
            pltpu.sync_copy(rows_v, agg_sh.at[idx_d.at[b]], add=True)

        plsc.subcore_barrier()
        off = s * ROWS_PER_TEC
        pltpu.sync_copy(agg_sh.at[pl.ds(off, ROWS_PER_TEC)],
                        out.at[pl.ds(off, ROWS_PER_TEC)])
        plsc.subcore_barrier()

    @pl.when(c == 0)
    def _():
        one_pass(hs_q0, out_q0)
        one_pass(hs_q1, out_q1)

    @pl.when(c == 1)
    def _():
        one_pass(hs_q2, out_q2)
        one_pass(hs_q3, out_q3)


def _sc_agg(hs_q, src_p, dst_p):
    mesh = plsc.VectorSubcoreMesh(core_axis_name="c", subcore_axis_name="s")
    qshape = jax.ShapeDtypeStruct((N_PAD, QF), jnp.float32)
    return pl.kernel(
        _sc_agg_body,
        out_type=(qshape,) * 4,
        mesh=mesh,
        scratch_types=[
            pltpu.VMEM((NB, EDGE_BLK), jnp.int32),
            pltpu.VMEM((NB, EDGE_BLK), jnp.int32),
            pltpu.VMEM((EDGE_BLK, QF), jnp.float32),
            pltpu.VMEM((64, QF), jnp.float32),
            pltpu.VMEM_SHARED((N_PAD, QF), jnp.float32),
            pltpu.SemaphoreType.DMA,
        ],
        compiler_params=pltpu.CompilerParams(needs_layout_passes=False,
                                             use_tc_tiling_on_sc=False),
    )(*hs_q, src_p, dst_p)


def _split_store(hs, outs):
    for q in range(4):
        outs[q][...] = hs[:, q * QF:(q + 1) * QF]


def _tc_embed_body(x_ref, w_ref, hist_ref, *outs):
    h = jnp.dot(x_ref[...], w_ref[...], preferred_element_type=jnp.float32)
    deg = jnp.sum(hist_ref[0], axis=1, keepdims=True)
    ns = lax.rsqrt(jnp.maximum(deg, 1.0))
    _split_store(h * ns, outs)


_QSPECS = [pl.BlockSpec((BN, QF), lambda i: (i, 0)) for _ in range(4)]
_QSHAPES = [jax.ShapeDtypeStruct((N_PAD, QF), jnp.float32) for _ in range(4)]


def _tc_embed(x, w0, hists_t):
    return pl.pallas_call(
        _tc_embed_body,
        grid=(N // BN,),
        in_specs=[
            pl.BlockSpec((BN, F), lambda i: (i, 0)),
            pl.BlockSpec((F, F), lambda i: (0, 0)),
            pl.BlockSpec((NSC, BN, NTEC), lambda i: (0, i, 0)),
        ],
        out_specs=_QSPECS,
        out_shape=_QSHAPES,
    )(x, w0, hists_t)


def _layer_common(aggs, hist_ref, w_ref, b_ref):
    a = jnp.concatenate([q[...] for q in aggs], axis=1)
    deg_in = jnp.sum(hist_ref[1], axis=1, keepdims=True)
    nd = lax.rsqrt(jnp.maximum(deg_in, 1.0))
    h = jnp.dot(a * nd, w_ref[...], preferred_element_type=jnp.float32)
    return jnp.maximum(h + b_ref[...], 0.0)


def _tc_layer_body(a0, a1, a2, a3, hist_ref, w_ref, b_ref, *outs):
    h = _layer_common((a0, a1, a2, a3), hist_ref, w_ref, b_ref)
    deg_out = jnp.sum(hist_ref[0], axis=1, keepdims=True)
    ns = lax.rsqrt(jnp.maximum(deg_out, 1.0))
    _split_store(h * ns, outs)


def _tc_layer(aggs, hists_t, w, b):
    return pl.pallas_call(
        _tc_layer_body,
        grid=(N // BN,),
        in_specs=_QSPECS + [
            pl.BlockSpec((NSC, BN, NTEC), lambda i: (0, i, 0)),
            pl.BlockSpec((F, F), lambda i: (0, 0)),
            pl.BlockSpec((1, F), lambda i: (0, 0)),
        ],
        out_specs=_QSPECS,
        out_shape=_QSHAPES,
    )(*aggs, hists_t, w, b)


def _tc_final_body(a0, a1, a2, a3, hist_ref, w_ref, b_ref, out_ref):
    i = pl.program_id(0)

    @pl.when(i == 0)
    def _():
        out_ref[...] = jnp.full((1, F), -jnp.inf, jnp.float32)

    h = _layer_common((a0, a1, a2, a3), hist_ref, w_ref, b_ref)
    out_ref[...] = jnp.maximum(out_ref[...], jnp.max(h, axis=0, keepdims=True))


def _tc_final(aggs, hists_t, w, b):
    return pl.pallas_call(
        _tc_final_body,
        grid=(N // BN,),
        in_specs=_QSPECS + [
            pl.BlockSpec((NSC, BN, NTEC), lambda i: (0, i, 0)),
            pl.BlockSpec((F, F), lambda i: (0, 0)),
            pl.BlockSpec((1, F), lambda i: (0, 0)),
        ],
        out_specs=pl.BlockSpec((1, F), lambda i: (0, 0)),
        out_shape=jax.ShapeDtypeStruct((1, F), jnp.float32),
        compiler_params=pltpu.CompilerParams(
            dimension_semantics=("arbitrary",)),
    )(*aggs, hists_t, w, b)


@jax.jit
def kernel(x, edge_index, W0, W1, b1, W2, b2):
    src = edge_index[0]
    dst = edge_index[1]
    pad = jnp.full((E_PAD - E,), N, jnp.int32)
    src_p = jnp.concatenate([src, pad]).reshape(NTEC, NB, EDGE_BLK)
    dst_p = jnp.concatenate([dst, pad]).reshape(NTEC, NB, EDGE_BLK)

    hists = _sc_degree(src_p, dst_p)
    hists_t = hists.transpose(0, 2, 1)

    hs_q = _tc_embed(x, W0, hists_t)
    agg_q = _sc_agg(hs_q, src_p, dst_p)
    hs_q = _tc_layer(agg_q, hists_t, W1, b1.reshape(1, F))
    agg_q = _sc_agg(hs_q, src_p, dst_p)
    out = _tc_final(agg_q, hists_t, W2, b2.reshape(1, F))
    return out.reshape(F)
